# baseline (speedup 1.0000x reference)
import functools

import jax
import jax.numpy as jnp
from jax import lax
from jax.experimental import pallas as pl
from jax.experimental.pallas import tpu as pltpu
from jax.experimental.pallas import tpu_sc as plsc

N_POS = 128
N_NEG = 768
N_ALL = N_POS + N_NEG
TEMPERATURE = 0.1
C = 192
H = 224
W = 224
HW = H * W
NBV = 4
TOT = NBV * HW

BAND = 8 * W
BPI = H // 8
NBANDS = NBV * BPI
WVB = W // 16

NC = 2
NS = 16
NCG = C // 16
PIX_C = N_ALL // NC
CHUNKS = PIX_C // 16


def _band_to_img(band):
    bv = band // BPI
    return bv % 2, bv // 2, band % BPI


def _scan_list(label_hbm, lab_band, buf, n_need, n_slack, want_pos):
    zeros = jnp.zeros((16,), jnp.int32)
    for i in range((n_need + n_slack) // 16):
        buf[pl.ds(i * 16, 16)] = zeros

    lane = lax.iota(jnp.int32, 16)
    cap = jnp.int32(n_need)

    def band_body(carry):
        blk, cnt = carry
        b, v, hb = _band_to_img(blk)
        pltpu.sync_copy(label_hbm.at[b, v, pl.ds(hb * 8, 8), :], lab_band)
        g0 = blk * BAND

        def row_body(carry):
            r, cnt = carry
            for wv in range(WVB):
                val = lab_band[r, pl.ds(wv * 16, 16)]
                idxv = (g0 + r * W + wv * 16) + lane
                if want_pos:
                    m = val > 0.0
                else:
                    m = (1.0 - val) > 0.0
                pc = plsc.all_reduce_population_count(m)[0]
                plsc.store_compressed(
                    buf.at[pl.ds(jnp.minimum(cnt, cap), 16)], idxv, mask=m)
                cnt = cnt + pc
            return r + 1, cnt

        def row_cond(carry):
            r, cnt = carry
            return (r < 8) & (cnt < n_need)

        _, cnt = lax.while_loop(row_cond, row_body, (jnp.int32(0), cnt))
        return blk + 1, cnt

    def band_cond(carry):
        blk, cnt = carry
        return (blk < NBANDS) & (cnt < n_need)

    lax.while_loop(band_cond, band_body, (jnp.int32(0), jnp.int32(0)))


def _sc_body(label_hbm, pred_hbm, emb_out, lab_band, pos_buf, neg_buf, samp,
             idx_v, band_v, sub_v, w_v, bandbuf, emb_t):
    cid = lax.axis_index("c")
    sid = lax.axis_index("s")

    def fetch(band):
        b, v, hb = _band_to_img(band)
        pltpu.sync_copy(
            pred_hbm.at[b, v, pl.ds(sid * 16, 16), pl.ds(hb * 8, 8), :],
            bandbuf)

    @pl.when(sid == 14)
    def _scan_pos():
        _scan_list(label_hbm, lab_band, pos_buf, N_POS, 32, True)
        pltpu.sync_copy(pos_buf.at[pl.ds(0, N_POS)], samp.at[pl.ds(0, N_POS)])

    @pl.when(sid == 15)
    def _scan_neg():
        _scan_list(label_hbm, lab_band, neg_buf, N_NEG, 16, False)
        pltpu.sync_copy(neg_buf.at[pl.ds(0, N_NEG)],
                        samp.at[pl.ds(N_POS, N_NEG)])

    @pl.when(sid < NCG)
    def _prefetch():
        fetch(jnp.int32(0))

    plsc.subcore_barrier()

    @pl.when(sid < NCG)
    def _extract():
        pltpu.sync_copy(samp.at[pl.ds(cid * PIX_C, PIX_C)], idx_v)

        def meta_body(i, _):
            t = idx_v[pl.ds(i * 16, 16)]
            band_v[pl.ds(i * 16, 16)] = t // BAND
            r = t % BAND
            sub_v[pl.ds(i * 16, 16)] = r // W
            w_v[pl.ds(i * 16, 16)] = r % W
            return 0

        lax.fori_loop(0, CHUNKS, meta_body, 0)

        c_splat = [jnp.full((16,), c, jnp.int32) for c in range(16)]
        lane = lax.iota(jnp.int32, 16)

        def chunk_body(ci, cur_band):
            bc = band_v[pl.ds(ci * 16, 16)]
            sc = sub_v[pl.ds(ci * 16, 16)]
            wc = w_v[pl.ds(ci * 16, 16)]
            b0 = bc[0]
            uniform = plsc.all_reduce_population_count(bc == b0)[0] == 16

            def fast(cur_band):
                @pl.when(b0 != cur_band)
                def _():
                    fetch(b0)

                for c in range(16):
                    vals = plsc.load_gather(bandbuf, [c_splat[c], sc, wc])
                    emb_t[c, pl.ds(ci * 16, 16)] = vals
                return b0

            def slow(cur_band):
                bmin = jnp.min(bc)
                bmax = jnp.max(bc)
                j_idx = ci * 16 + lane

                def wbody(carry):
                    band, cur = carry

                    @pl.when(band != cur)
                    def _():
                        fetch(band)

                    m = bc == band
                    for c in range(16):
                        vals = plsc.load_gather(bandbuf, [c_splat[c], sc, wc],
                                                mask=m)
                        plsc.store_scatter(emb_t, [c_splat[c], j_idx], vals,
                                           mask=m)
                    return band + 1, band

                def wcond(carry):
                    return carry[0] <= bmax

                _, _ = lax.while_loop(wcond, wbody, (bmin, cur_band))
                return bmax

            return lax.cond(uniform, fast, slow, cur_band)

        lax.fori_loop(0, CHUNKS, chunk_body, jnp.int32(0))

        pltpu.sync_copy(emb_t, emb_out.at[cid, sid])


@jax.jit
def _sc_sample_gather(pred5d, label4d):
    mesh = plsc.VectorSubcoreMesh(core_axis_name="c", subcore_axis_name="s",
                                  num_cores=NC, num_subcores=NS)
    return pl.kernel(
        _sc_body,
        out_type=jax.ShapeDtypeStruct((NC, NCG, 16, PIX_C), jnp.float32),
        mesh=mesh,
        compiler_params=pltpu.CompilerParams(use_tc_tiling_on_sc=True,
                                             needs_layout_passes=False),
        scratch_types=[
            pltpu.VMEM((8, W), jnp.float32),
            pltpu.VMEM((160,), jnp.int32),
            pltpu.VMEM((784,), jnp.int32),
            pltpu.VMEM_SHARED((1024,), jnp.int32),
            pltpu.VMEM((PIX_C,), jnp.int32),
            pltpu.VMEM((PIX_C,), jnp.int32),
            pltpu.VMEM((PIX_C,), jnp.int32),
            pltpu.VMEM((PIX_C,), jnp.int32),
            pltpu.VMEM((16, 8, W), jnp.float32),
            pltpu.VMEM((16, PIX_C), jnp.float32),
        ],
    )(label4d, pred5d)


def _loss_body(e4_ref, out_ref):
    e4 = e4_ref[...]
    er = e4.reshape(NC, C, PIX_C)
    n = jnp.maximum(jnp.sqrt(jnp.sum(er * er, axis=1)), 1e-8)
    posr = er[0, :, :N_POS]
    s = lax.dot_general(posr, er, (((0,), (1,)), ((), ())),
                        preferred_element_type=jnp.float32,
                        precision=lax.Precision.HIGHEST)
    npos = n[0, :N_POS]
    e = jnp.exp(s / (npos[:, None, None] * n[None, :, :]) / TEMPERATURE)
    ii = lax.broadcasted_iota(jnp.int32, (N_POS, N_POS), 0)
    jj = lax.broadcasted_iota(jnp.int32, (N_POS, N_POS), 1)
    epp = jnp.where(ii == jj, 0.0, e[:, 0, :N_POS])
    pos_row = jnp.sum(epp, axis=1)
    neg_row = jnp.sum(e[:, 0, N_POS:], axis=1) + jnp.sum(e[:, 1, :], axis=1)
    nll = -jnp.mean(jnp.log(pos_row / (pos_row + neg_row)))
    out_ref[...] = jnp.full((1, 1), nll, jnp.float32)


@jax.jit
def _loss(emb4):
    return pl.pallas_call(
        _loss_body,
        out_shape=jax.ShapeDtypeStruct((1, 1), jnp.float32),
    )(emb4)


def kernel(predict_seg_map, real_label):
    emb4 = _sc_sample_gather(predict_seg_map, real_label)
    return _loss(emb4)[0, 0]


# speedup vs baseline: 1.0939x; 1.0939x over previous
import functools

import jax
import jax.numpy as jnp
from jax import lax
from jax.experimental import pallas as pl
from jax.experimental.pallas import tpu as pltpu
from jax.experimental.pallas import tpu_sc as plsc

N_POS = 128
N_NEG = 768
N_ALL = N_POS + N_NEG
TEMPERATURE = 0.1
C = 192
H = 224
W = 224
HW = H * W
NBV = 4
TOT = NBV * HW

BAND = 8 * W
BPI = H // 8
NBANDS = NBV * BPI
WVB = W // 16

NC = 2
NS = 16
NCG = C // 16
PIX_C = N_ALL // NC
CHUNKS = PIX_C // 16


def _band_to_img(band):
    bv = band // BPI
    return bv % 2, bv // 2, band % BPI


def _scan_list(label_hbm, lab_band, buf, n_need, n_slack, want_pos):
    zeros = jnp.zeros((16,), jnp.int32)
    for i in range((n_need + n_slack) // 16):
        buf[pl.ds(i * 16, 16)] = zeros

    lane = lax.iota(jnp.int32, 16)
    cap = jnp.int32(n_need)

    def band_body(carry):
        blk, cnt = carry
        b, v, hb = _band_to_img(blk)
        pltpu.sync_copy(label_hbm.at[b, v, pl.ds(hb * 8, 8), :], lab_band)

        def row_body(carry):
            r, cnt = carry
            for wv in range(WVB):
                val = lab_band[r, pl.ds(wv * 16, 16)]
                idxv = ((blk << 11) | (r << 8) | (wv * 16)) + lane
                if want_pos:
                    m = val > 0.0
                else:
                    m = (1.0 - val) > 0.0
                pc = plsc.all_reduce_population_count(m)[0]
                plsc.store_compressed(
                    buf.at[pl.ds(jnp.minimum(cnt, cap), 16)], idxv, mask=m)
                cnt = cnt + pc
            return r + 1, cnt

        def row_cond(carry):
            r, cnt = carry
            return (r < 8) & (cnt < n_need)

        _, cnt = lax.while_loop(row_cond, row_body, (jnp.int32(0), cnt))
        return blk + 1, cnt

    def band_cond(carry):
        blk, cnt = carry
        return (blk < NBANDS) & (cnt < n_need)

    lax.while_loop(band_cond, band_body, (jnp.int32(0), jnp.int32(0)))


def _sc_body(label_hbm, pred_hbm, emb_out, lab_band, pos_buf, neg_buf, samp,
             idx_v, band_v, sub_v, w_v, bandbuf, emb_t):
    cid = lax.axis_index("c")
    sid = lax.axis_index("s")

    def fetch(band):
        b, v, hb = _band_to_img(band)
        pltpu.sync_copy(
            pred_hbm.at[b, v, pl.ds(sid * 16, 16), pl.ds(hb * 8, 8), :],
            bandbuf)

    @pl.when(sid == 14)
    def _scan_pos():
        _scan_list(label_hbm, lab_band, pos_buf, N_POS, 32, True)
        pltpu.sync_copy(pos_buf.at[pl.ds(0, N_POS)], samp.at[pl.ds(0, N_POS)])

    @pl.when(sid == 15)
    def _scan_neg():
        _scan_list(label_hbm, lab_band, neg_buf, N_NEG, 16, False)
        pltpu.sync_copy(neg_buf.at[pl.ds(0, N_NEG)],
                        samp.at[pl.ds(N_POS, N_NEG)])

    @pl.when(sid < NCG)
    def _prefetch():
        fetch(jnp.int32(0))

    plsc.subcore_barrier()

    @pl.when(sid < NCG)
    def _extract():
        pltpu.sync_copy(samp.at[pl.ds(cid * PIX_C, PIX_C)], idx_v)

        def meta_body(i, _):
            t = idx_v[pl.ds(i * 16, 16)]
            band_v[pl.ds(i * 16, 16)] = t >> 11
            sub_v[pl.ds(i * 16, 16)] = (t >> 8) & 7
            w_v[pl.ds(i * 16, 16)] = t & 255
            return 0

        lax.fori_loop(0, CHUNKS, meta_body, 0)

        c_splat = [jnp.full((16,), c, jnp.int32) for c in range(16)]
        lane = lax.iota(jnp.int32, 16)

        def chunk_body(ci, cur_band):
            bc = band_v[pl.ds(ci * 16, 16)]
            sc = sub_v[pl.ds(ci * 16, 16)]
            wc = w_v[pl.ds(ci * 16, 16)]
            b0 = bc[0]
            uniform = plsc.all_reduce_population_count(bc == b0)[0] == 16

            def fast(cur_band):
                @pl.when(b0 != cur_band)
                def _():
                    fetch(b0)

                for c in range(16):
                    vals = plsc.load_gather(bandbuf, [c_splat[c], sc, wc])
                    emb_t[c, pl.ds(ci * 16, 16)] = vals
                return b0

            def slow(cur_band):
                bmin = jnp.min(bc)
                bmax = jnp.max(bc)
                j_idx = ci * 16 + lane

                def wbody(carry):
                    band, cur = carry

                    @pl.when(band != cur)
                    def _():
                        fetch(band)

                    m = bc == band
                    for c in range(16):
                        vals = plsc.load_gather(bandbuf, [c_splat[c], sc, wc],
                                                mask=m)
                        plsc.store_scatter(emb_t, [c_splat[c], j_idx], vals,
                                           mask=m)
                    return band + 1, band

                def wcond(carry):
                    return carry[0] <= bmax

                _, _ = lax.while_loop(wcond, wbody, (bmin, cur_band))
                return bmax

            return lax.cond(uniform, fast, slow, cur_band)

        lax.fori_loop(0, CHUNKS, chunk_body, jnp.int32(0))

        pltpu.sync_copy(emb_t, emb_out.at[cid, sid])


@jax.jit
def _sc_sample_gather(pred5d, label4d):
    mesh = plsc.VectorSubcoreMesh(core_axis_name="c", subcore_axis_name="s",
                                  num_cores=NC, num_subcores=NS)
    return pl.kernel(
        _sc_body,
        out_type=jax.ShapeDtypeStruct((NC, NCG, 16, PIX_C), jnp.float32),
        mesh=mesh,
        compiler_params=pltpu.CompilerParams(use_tc_tiling_on_sc=True,
                                             needs_layout_passes=False),
        scratch_types=[
            pltpu.VMEM((8, W), jnp.float32),
            pltpu.VMEM((160,), jnp.int32),
            pltpu.VMEM((784,), jnp.int32),
            pltpu.VMEM_SHARED((1024,), jnp.int32),
            pltpu.VMEM((PIX_C,), jnp.int32),
            pltpu.VMEM((PIX_C,), jnp.int32),
            pltpu.VMEM((PIX_C,), jnp.int32),
            pltpu.VMEM((PIX_C,), jnp.int32),
            pltpu.VMEM((16, 8, W), jnp.float32),
            pltpu.VMEM((16, PIX_C), jnp.float32),
        ],
    )(label4d, pred5d)


def _loss_body(e4_ref, out_ref):
    e4 = e4_ref[...]
    er = e4.reshape(NC, C, PIX_C)
    n = jnp.maximum(jnp.sqrt(jnp.sum(er * er, axis=1)), 1e-8)
    posr = er[0, :, :N_POS]
    s = lax.dot_general(posr, er, (((0,), (1,)), ((), ())),
                        preferred_element_type=jnp.float32,
                        precision=lax.Precision.HIGHEST)
    npos = n[0, :N_POS]
    e = jnp.exp(s / (npos[:, None, None] * n[None, :, :]) / TEMPERATURE)
    ii = lax.broadcasted_iota(jnp.int32, (N_POS, N_POS), 0)
    jj = lax.broadcasted_iota(jnp.int32, (N_POS, N_POS), 1)
    epp = jnp.where(ii == jj, 0.0, e[:, 0, :N_POS])
    pos_row = jnp.sum(epp, axis=1)
    neg_row = jnp.sum(e[:, 0, N_POS:], axis=1) + jnp.sum(e[:, 1, :], axis=1)
    nll = -jnp.mean(jnp.log(pos_row / (pos_row + neg_row)))
    out_ref[...] = jnp.full((1, 1), nll, jnp.float32)


@jax.jit
def _loss(emb4):
    return pl.pallas_call(
        _loss_body,
        out_shape=jax.ShapeDtypeStruct((1, 1), jnp.float32),
    )(emb4)


def kernel(predict_seg_map, real_label):
    emb4 = _sc_sample_gather(predict_seg_map, real_label)
    return _loss(emb4)[0, 0]


# speedup vs baseline: 1.1092x; 1.0140x over previous
import functools

import jax
import jax.numpy as jnp
from jax import lax
from jax.experimental import pallas as pl
from jax.experimental.pallas import tpu as pltpu
from jax.experimental.pallas import tpu_sc as plsc

N_POS = 128
N_NEG = 768
N_ALL = N_POS + N_NEG
TEMPERATURE = 0.1
C = 192
H = 224
W = 224
HW = H * W
NBV = 4
TOT = NBV * HW

BAND = 8 * W
BPI = H // 8
NBANDS = NBV * BPI
WVB = W // 16

NC = 2
NS = 16
NCG = C // 16
PIX_C = N_ALL // NC
CHUNKS = PIX_C // 16


def _band_to_img(band):
    bv = band // BPI
    return bv % 2, bv // 2, band % BPI


def _scan_list(label_hbm, lab_band, buf, n_need, n_slack, want_pos):
    zeros = jnp.zeros((16,), jnp.int32)
    for i in range((n_need + n_slack) // 16):
        buf[pl.ds(i * 16, 16)] = zeros

    lane = lax.iota(jnp.int32, 16)
    cap = jnp.int32(n_need)

    def band_body(carry):
        blk, cnt = carry
        b, v, hb = _band_to_img(blk)
        pltpu.sync_copy(label_hbm.at[b, v, pl.ds(hb * 8, 8), :], lab_band)

        def row_body(carry):
            r, cnt = carry
            for wv in range(WVB):
                val = lab_band[r, pl.ds(wv * 16, 16)]
                idxv = ((blk << 11) | (r << 8) | (wv * 16)) + lane
                if want_pos:
                    m = val > 0.0
                else:
                    m = (1.0 - val) > 0.0
                pc = plsc.all_reduce_population_count(m)[0]
                plsc.store_compressed(
                    buf.at[pl.ds(jnp.minimum(cnt, cap), 16)], idxv, mask=m)
                cnt = cnt + pc
            return r + 1, cnt

        def row_cond(carry):
            r, cnt = carry
            return (r < 8) & (cnt < n_need)

        _, cnt = lax.while_loop(row_cond, row_body, (jnp.int32(0), cnt))
        return blk + 1, cnt

    def band_cond(carry):
        blk, cnt = carry
        return (blk < NBANDS) & (cnt < n_need)

    lax.while_loop(band_cond, band_body, (jnp.int32(0), jnp.int32(0)))


def _sc_body(label_hbm, pred_hbm, emb_out, lab_band, pos_buf, neg_buf, samp,
             idx_v, band_v, sub_v, w_v, bandbuf, emb_t):
    cid = lax.axis_index("c")
    sid = lax.axis_index("s")

    def fetch(band):
        b, v, hb = _band_to_img(band)
        pltpu.sync_copy(
            pred_hbm.at[b, v, pl.ds(sid * 16, 16), pl.ds(hb * 8, 8), :],
            bandbuf)

    @pl.when(sid == 14)
    def _scan_pos():
        _scan_list(label_hbm, lab_band, pos_buf, N_POS, 32, True)
        pltpu.sync_copy(pos_buf.at[pl.ds(0, N_POS)], samp.at[pl.ds(0, N_POS)])

    @pl.when(sid == 15)
    def _scan_neg():
        _scan_list(label_hbm, lab_band, neg_buf, N_NEG, 16, False)
        pltpu.sync_copy(neg_buf.at[pl.ds(0, N_NEG)],
                        samp.at[pl.ds(N_POS, N_NEG)])

    @pl.when(sid < NCG)
    def _prefetch():
        fetch(jnp.int32(0))

    plsc.subcore_barrier()

    @pl.when(sid < NCG)
    def _extract():
        pltpu.sync_copy(samp.at[pl.ds(cid * PIX_C, PIX_C)], idx_v)

        def meta_body(i, _):
            t = idx_v[pl.ds(i * 16, 16)]
            band_v[pl.ds(i * 16, 16)] = t >> 11
            sub_v[pl.ds(i * 16, 16)] = (t >> 8) & 7
            w_v[pl.ds(i * 16, 16)] = t & 255
            return 0

        lax.fori_loop(0, CHUNKS, meta_body, 0)

        c_splat = [jnp.full((16,), c, jnp.int32) for c in range(16)]
        lane = lax.iota(jnp.int32, 16)

        def chunk_body(ci, cur_band):
            bc = band_v[pl.ds(ci * 16, 16)]
            sc = sub_v[pl.ds(ci * 16, 16)]
            wc = w_v[pl.ds(ci * 16, 16)]
            b0 = bc[0]
            uniform = plsc.all_reduce_population_count(bc == b0)[0] == 16

            def fast(cur_band):
                @pl.when(b0 != cur_band)
                def _():
                    fetch(b0)

                for c in range(16):
                    vals = plsc.load_gather(bandbuf, [c_splat[c], sc, wc])
                    emb_t[c, pl.ds(ci * 16, 16)] = vals
                return b0

            def slow(cur_band):
                bmin = jnp.min(bc)
                bmax = jnp.max(bc)
                j_idx = ci * 16 + lane

                def wbody(carry):
                    band, cur = carry

                    @pl.when(band != cur)
                    def _():
                        fetch(band)

                    m = bc == band
                    for c in range(16):
                        vals = plsc.load_gather(bandbuf, [c_splat[c], sc, wc],
                                                mask=m)
                        plsc.store_scatter(emb_t, [c_splat[c], j_idx], vals,
                                           mask=m)
                    return band + 1, band

                def wcond(carry):
                    return carry[0] <= bmax

                _, _ = lax.while_loop(wcond, wbody, (bmin, cur_band))
                return bmax

            return lax.cond(uniform, fast, slow, cur_band)

        lax.fori_loop(0, CHUNKS, chunk_body, jnp.int32(0))

        pltpu.sync_copy(emb_t, emb_out.at[cid, sid])


@jax.jit
def _sc_sample_gather(pred5d, label4d):
    mesh = plsc.VectorSubcoreMesh(core_axis_name="c", subcore_axis_name="s",
                                  num_cores=NC, num_subcores=NS)
    return pl.kernel(
        _sc_body,
        out_type=jax.ShapeDtypeStruct((NC, NCG, 16, PIX_C), jnp.float32),
        mesh=mesh,
        compiler_params=pltpu.CompilerParams(use_tc_tiling_on_sc=True,
                                             needs_layout_passes=False),
        scratch_types=[
            pltpu.VMEM((8, W), jnp.float32),
            pltpu.VMEM((160,), jnp.int32),
            pltpu.VMEM((784,), jnp.int32),
            pltpu.VMEM_SHARED((1024,), jnp.int32),
            pltpu.VMEM((PIX_C,), jnp.int32),
            pltpu.VMEM((PIX_C,), jnp.int32),
            pltpu.VMEM((PIX_C,), jnp.int32),
            pltpu.VMEM((PIX_C,), jnp.int32),
            pltpu.VMEM((16, 8, W), jnp.float32),
            pltpu.VMEM((16, PIX_C), jnp.float32),
        ],
    )(label4d, pred5d)


def _loss_body(e4_ref, out_ref):
    e4 = e4_ref[...]
    er = e4.reshape(NC, C, PIX_C)
    n = jnp.maximum(jnp.sqrt(jnp.sum(er * er, axis=1)), 1e-8)
    posr = er[0, :, :N_POS]
    s = lax.dot_general(posr, er, (((0,), (1,)), ((), ())),
                        preferred_element_type=jnp.float32)
    npos = n[0, :N_POS]
    e = jnp.exp(s / (npos[:, None, None] * n[None, :, :]) / TEMPERATURE)
    ii = lax.broadcasted_iota(jnp.int32, (N_POS, N_POS), 0)
    jj = lax.broadcasted_iota(jnp.int32, (N_POS, N_POS), 1)
    epp = jnp.where(ii == jj, 0.0, e[:, 0, :N_POS])
    pos_row = jnp.sum(epp, axis=1)
    neg_row = jnp.sum(e[:, 0, N_POS:], axis=1) + jnp.sum(e[:, 1, :], axis=1)
    nll = -jnp.mean(jnp.log(pos_row / (pos_row + neg_row)))
    out_ref[...] = jnp.full((1, 1), nll, jnp.float32)


@jax.jit
def _loss(emb4):
    return pl.pallas_call(
        _loss_body,
        out_shape=jax.ShapeDtypeStruct((1, 1), jnp.float32),
    )(emb4)


def kernel(predict_seg_map, real_label):
    emb4 = _sc_sample_gather(predict_seg_map, real_label)
    return _loss(emb4)[0, 0]


# speedup vs baseline: 1.1121x; 1.0026x over previous
import jax
import jax.numpy as jnp
from jax import lax
from jax.experimental import pallas as pl
from jax.experimental.pallas import tpu as pltpu
from jax.experimental.pallas import tpu_sc as plsc

N_POS = 128
N_NEG = 768
N_ALL = N_POS + N_NEG
TEMPERATURE = 0.1
C = 192
H = 224
W = 224
HW = H * W
NBV = 4
TOT = NBV * HW

BAND = 8 * W
BPI = H // 8
NBANDS = NBV * BPI
WVB = W // 16

NC = 2
NS = 16
NCG = C // 16
PIX_C = N_ALL // NC
CHUNKS = PIX_C // 16


def _band_to_img(band):
    bv = band // BPI
    return bv % 2, bv // 2, band % BPI


def _scan_list(label_hbm, lab_band, buf, n_need, n_slack, want_pos):
    zeros = jnp.zeros((16,), jnp.int32)
    for i in range((n_need + n_slack) // 16):
        buf[pl.ds(i * 16, 16)] = zeros

    lane = lax.iota(jnp.int32, 16)
    cap = jnp.int32(n_need)

    def band_body(carry):
        blk, cnt = carry
        b, v, hb = _band_to_img(blk)
        pltpu.sync_copy(label_hbm.at[b, v, pl.ds(hb * 8, 8), :], lab_band)

        def row_body(carry):
            r, cnt = carry
            for wv in range(WVB):
                val = lab_band[r, pl.ds(wv * 16, 16)]
                idxv = ((blk << 11) | (r << 8) | (wv * 16)) + lane
                if want_pos:
                    m = val > 0.0
                else:
                    m = (1.0 - val) > 0.0
                pc = plsc.all_reduce_population_count(m)[0]
                plsc.store_compressed(
                    buf.at[pl.ds(jnp.minimum(cnt, cap), 16)], idxv, mask=m)
                cnt = cnt + pc
            return r + 1, cnt

        def row_cond(carry):
            r, cnt = carry
            return (r < 8) & (cnt < n_need)

        _, cnt = lax.while_loop(row_cond, row_body, (jnp.int32(0), cnt))
        return blk + 1, cnt

    def band_cond(carry):
        blk, cnt = carry
        return (blk < NBANDS) & (cnt < n_need)

    lax.while_loop(band_cond, band_body, (jnp.int32(0), jnp.int32(0)))


def _sc_body(label_hbm, pred_hbm, emb_out, lab_band, pos_buf, neg_buf, samp,
             idx_v, band_v, sub_v, w_v, bandbuf, emb_t):
    cid = lax.axis_index("c")
    sid = lax.axis_index("s")

    def fetch(band):
        b, v, hb = _band_to_img(band)
        pltpu.sync_copy(
            pred_hbm.at[b, v, pl.ds(sid * 16, 16), pl.ds(hb * 8, 8), :],
            bandbuf)

    @pl.when(sid == 14)
    def _scan_pos():
        _scan_list(label_hbm, lab_band, pos_buf, N_POS, 32, True)
        pltpu.sync_copy(pos_buf.at[pl.ds(0, N_POS)], samp.at[pl.ds(0, N_POS)])

    @pl.when(sid == 15)
    def _scan_neg():
        _scan_list(label_hbm, lab_band, neg_buf, N_NEG, 16, False)
        pltpu.sync_copy(neg_buf.at[pl.ds(0, N_NEG)],
                        samp.at[pl.ds(N_POS, N_NEG)])

    @pl.when(sid < NCG)
    def _prefetch():
        fetch(jnp.int32(0))

    plsc.subcore_barrier()

    @pl.when(sid < NCG)
    def _extract():
        pltpu.sync_copy(samp.at[pl.ds(cid * PIX_C, PIX_C)], idx_v)

        def meta_body(i, _):
            t = idx_v[pl.ds(i * 16, 16)]
            band_v[pl.ds(i * 16, 16)] = t >> 11
            sub_v[pl.ds(i * 16, 16)] = (t >> 8) & 7
            w_v[pl.ds(i * 16, 16)] = t & 255
            return 0

        lax.fori_loop(0, CHUNKS, meta_body, 0)

        c_splat = [jnp.full((16,), c, jnp.int32) for c in range(16)]
        lane = lax.iota(jnp.int32, 16)

        def chunk_body(ci, cur_band):
            bc = band_v[pl.ds(ci * 16, 16)]
            sc = sub_v[pl.ds(ci * 16, 16)]
            wc = w_v[pl.ds(ci * 16, 16)]
            b0 = bc[0]
            uniform = plsc.all_reduce_population_count(bc == b0)[0] == 16

            def fast(cur_band):
                @pl.when(b0 != cur_band)
                def _():
                    fetch(b0)

                for c in range(16):
                    vals = plsc.load_gather(bandbuf, [c_splat[c], sc, wc])
                    emb_t[c, pl.ds(ci * 16, 16)] = vals
                return b0

            def slow(cur_band):
                bmin = jnp.min(bc)
                bmax = jnp.max(bc)
                j_idx = ci * 16 + lane

                def wbody(carry):
                    band, cur = carry

                    @pl.when(band != cur)
                    def _():
                        fetch(band)

                    m = bc == band
                    for c in range(16):
                        vals = plsc.load_gather(bandbuf, [c_splat[c], sc, wc],
                                                mask=m)
                        plsc.store_scatter(emb_t, [c_splat[c], j_idx], vals,
                                           mask=m)
                    return band + 1, band

                def wcond(carry):
                    return carry[0] <= bmax

                _, _ = lax.while_loop(wcond, wbody, (bmin, cur_band))
                return bmax

            return lax.cond(uniform, fast, slow, cur_band)

        lax.fori_loop(0, CHUNKS, chunk_body, jnp.int32(0))

        pltpu.sync_copy(emb_t, emb_out.at[cid, sid])


@jax.jit
def _sc_sample_gather(pred5d, label4d):
    mesh = plsc.VectorSubcoreMesh(core_axis_name="c", subcore_axis_name="s",
                                  num_cores=NC, num_subcores=NS)
    return pl.kernel(
        _sc_body,
        out_type=jax.ShapeDtypeStruct((NC, NCG, 16, PIX_C), jnp.float32),
        mesh=mesh,
        compiler_params=pltpu.CompilerParams(use_tc_tiling_on_sc=True,
                                             needs_layout_passes=False),
        scratch_types=[
            pltpu.VMEM((8, W), jnp.float32),
            pltpu.VMEM((160,), jnp.int32),
            pltpu.VMEM((784,), jnp.int32),
            pltpu.VMEM_SHARED((1024,), jnp.int32),
            pltpu.VMEM((PIX_C,), jnp.int32),
            pltpu.VMEM((PIX_C,), jnp.int32),
            pltpu.VMEM((PIX_C,), jnp.int32),
            pltpu.VMEM((PIX_C,), jnp.int32),
            pltpu.VMEM((16, 8, W), jnp.float32),
            pltpu.VMEM((16, PIX_C), jnp.float32),
        ],
    )(label4d, pred5d)


def _loss_body(e4_ref, out_ref):
    e4 = e4_ref[...]
    er = e4.reshape(NC, C, PIX_C)
    n = jnp.maximum(jnp.sqrt(jnp.sum(er * er, axis=1)), 1e-8)
    posr = er[0, :, :N_POS]
    s = lax.dot_general(posr, er, (((0,), (1,)), ((), ())),
                        preferred_element_type=jnp.float32)
    npos = n[0, :N_POS]
    e = jnp.exp(s / (npos[:, None, None] * n[None, :, :]) / TEMPERATURE)
    ii = lax.broadcasted_iota(jnp.int32, (N_POS, N_POS), 0)
    jj = lax.broadcasted_iota(jnp.int32, (N_POS, N_POS), 1)
    epp = jnp.where(ii == jj, 0.0, e[:, 0, :N_POS])
    pos_row = jnp.sum(epp, axis=1)
    neg_row = jnp.sum(e[:, 0, N_POS:], axis=1) + jnp.sum(e[:, 1, :], axis=1)
    nll = -jnp.mean(jnp.log(pos_row / (pos_row + neg_row)))
    out_ref[...] = jnp.full((1, 1), nll, jnp.float32)


@jax.jit
def _loss(emb4):
    return pl.pallas_call(
        _loss_body,
        out_shape=jax.ShapeDtypeStruct((1, 1), jnp.float32),
    )(emb4)


def kernel(predict_seg_map, real_label):
    emb4 = _sc_sample_gather(predict_seg_map, real_label)
    return _loss(emb4)[0, 0]
